# Initial kernel scaffold; baseline (speedup 1.0000x reference)
import functools

import jax
import jax.numpy as jnp
from jax import lax
from jax.experimental import pallas as pl
from jax.experimental.pallas import tpu as pltpu
from jax.experimental.pallas import tpu_sc as plsc

NUM_GROUPS = 64
LANES = 16
NC = 2
NS = 16
NW = NC * NS


def _sort16_asc(v):
    return jnp.sort(v)


def _sort16_desc(v):
    return -jnp.sort(-v)


def _merge_asc(u, v):
    lo = jnp.minimum(u, v)
    hi = jnp.maximum(u, v)
    return _sort16_asc(lo), _sort16_asc(hi)


def _merge_desc(u, v):
    lo = jnp.minimum(u, v)
    hi = jnp.maximum(u, v)
    return _sort16_desc(hi), _sort16_desc(lo)


def _sort64(a0, a1, a2, a3):
    s0 = _sort16_asc(a0)
    s1 = _sort16_desc(a1)
    s2 = _sort16_asc(a2)
    s3 = _sort16_desc(a3)
    x0, x1 = _merge_asc(s0, s1)
    y0, y1 = _merge_desc(s2, s3)
    l0 = jnp.minimum(x0, y0)
    l1 = jnp.minimum(x1, y1)
    h0 = jnp.maximum(x0, y0)
    h1 = jnp.maximum(x1, y1)
    p0 = jnp.minimum(l0, l1)
    p1 = jnp.maximum(l0, l1)
    p2 = jnp.minimum(h0, h1)
    p3 = jnp.maximum(h0, h1)
    return (_sort16_asc(p0), _sort16_asc(p1),
            _sort16_asc(p2), _sort16_asc(p3))


def _make_sc_call(nrows, groups_per_row):
    rows_per_worker = nrows // NW
    mesh = plsc.VectorSubcoreMesh(core_axis_name="c", subcore_axis_name="s")

    @functools.partial(
        pl.kernel,
        mesh=mesh,
        out_type=jax.ShapeDtypeStruct((nrows, groups_per_row, 4, LANES),
                                      jnp.float32),
        scratch_types=[
            pltpu.VMEM((groups_per_row, 4, LANES), jnp.float32),
            pltpu.VMEM((groups_per_row, 4, LANES), jnp.float32),
        ],
    )
    def sc_sort(x_hbm, out_hbm, in_v, out_v):
        wid = lax.axis_index("c") * NS + lax.axis_index("s")

        def row_body(i, carry):
            row = wid * rows_per_worker + i
            pltpu.sync_copy(x_hbm.at[row], in_v)

            def group_body(g, c):
                a0 = in_v[g, 0]
                a1 = in_v[g, 1]
                a2 = in_v[g, 2]
                a3 = in_v[g, 3]
                r0, r1, r2, r3 = _sort64(a0, a1, a2, a3)
                out_v[g, 0] = r0
                out_v[g, 1] = r1
                out_v[g, 2] = r2
                out_v[g, 3] = r3
                return c

            lax.fori_loop(0, groups_per_row, group_body, 0)
            pltpu.sync_copy(out_v, out_hbm.at[row])
            return carry

        lax.fori_loop(0, rows_per_worker, row_body, 0)

    return sc_sort


def kernel(x):
    shape = x.shape
    num_features = shape[-1]
    group_size = num_features // NUM_GROUPS
    total_groups = x.size // group_size
    nrows = 1024
    groups_per_row = total_groups // nrows
    x2 = x.reshape(nrows, groups_per_row, 4, LANES)
    y2 = _make_sc_call(nrows, groups_per_row)(x2)
    return y2.reshape(shape)


# speedup vs baseline: 14.8543x; 14.8543x over previous
import functools

import jax
import jax.numpy as jnp
from jax import lax
from jax.experimental import pallas as pl
from jax.experimental.pallas import tpu as pltpu
from jax.experimental.pallas import tpu_sc as plsc

NUM_GROUPS = 64
LANES = 16
NC = 2
NS = 16
NW = NC * NS


def _sort16_asc(v):
    return jnp.sort(v)


def _sort16_desc(v):
    return -jnp.sort(-v)


def _merge_asc(u, v):
    lo = jnp.minimum(u, v)
    hi = jnp.maximum(u, v)
    return _sort16_asc(lo), _sort16_asc(hi)


def _merge_desc(u, v):
    lo = jnp.minimum(u, v)
    hi = jnp.maximum(u, v)
    return _sort16_desc(hi), _sort16_desc(lo)


def _sort64(a0, a1, a2, a3):
    s0 = _sort16_asc(a0)
    s1 = _sort16_desc(a1)
    s2 = _sort16_asc(a2)
    s3 = _sort16_desc(a3)
    x0, x1 = _merge_asc(s0, s1)
    y0, y1 = _merge_desc(s2, s3)
    l0 = jnp.minimum(x0, y0)
    l1 = jnp.minimum(x1, y1)
    h0 = jnp.maximum(x0, y0)
    h1 = jnp.maximum(x1, y1)
    p0 = jnp.minimum(l0, l1)
    p1 = jnp.maximum(l0, l1)
    p2 = jnp.minimum(h0, h1)
    p3 = jnp.maximum(h0, h1)
    return (_sort16_asc(p0), _sort16_asc(p1),
            _sort16_asc(p2), _sort16_asc(p3))


def _make_sc_call(nrows, groups_per_row):
    rows_per_worker = nrows // NW
    mesh = plsc.VectorSubcoreMesh(core_axis_name="c", subcore_axis_name="s")

    @functools.partial(
        pl.kernel,
        mesh=mesh,
        out_type=jax.ShapeDtypeStruct((nrows, groups_per_row, 4, LANES),
                                      jnp.float32),
        scratch_types=[
            pltpu.VMEM((groups_per_row, 4, LANES), jnp.float32),
            pltpu.VMEM((groups_per_row, 4, LANES), jnp.float32),
        ],
        compiler_params=pltpu.CompilerParams(
            needs_layout_passes=False, use_tc_tiling_on_sc=False),
    )
    def sc_sort(x_hbm, out_hbm, in_v, out_v):
        wid = lax.axis_index("c") * NS + lax.axis_index("s")

        def row_body(i, carry):
            row = wid * rows_per_worker + i
            pltpu.sync_copy(x_hbm.at[row], in_v)

            def group_body(g, c):
                a0 = in_v[g, 0]
                a1 = in_v[g, 1]
                a2 = in_v[g, 2]
                a3 = in_v[g, 3]
                r0, r1, r2, r3 = _sort64(a0, a1, a2, a3)
                out_v[g, 0] = r0
                out_v[g, 1] = r1
                out_v[g, 2] = r2
                out_v[g, 3] = r3
                return c

            lax.fori_loop(0, groups_per_row, group_body, 0)
            pltpu.sync_copy(out_v, out_hbm.at[row])
            return carry

        lax.fori_loop(0, rows_per_worker, row_body, 0)

    return sc_sort


def kernel(x):
    shape = x.shape
    num_features = shape[-1]
    group_size = num_features // NUM_GROUPS
    total_groups = x.size // group_size
    nrows = 1024
    groups_per_row = total_groups // nrows
    x2 = x.reshape(nrows, groups_per_row, 4, LANES)
    y2 = _make_sc_call(nrows, groups_per_row)(x2)
    return y2.reshape(shape)
